# baseline (speedup 1.0000x reference)
import functools

import jax
import jax.numpy as jnp
from jax import lax
from jax.experimental import pallas as pl
from jax.experimental.pallas import tpu as pltpu
from jax.experimental.pallas import tpu_sc as plsc


def _copy_body(w_hbm, o_hbm, w_vmem, in_sem, out_sem, *, B, K, CT):
    ins = [
        pltpu.make_async_copy(
            w_hbm.at[pl.ds(k * CT, CT), :],
            w_vmem.at[pl.ds(k * CT, CT), :],
            in_sem.at[k],
        )
        for k in range(K)
    ]
    for c in ins:
        c.start()
    outs = []
    for k in range(K):
        ins[k].wait()
        for b in range(B):
            c = pltpu.make_async_copy(
                w_vmem.at[pl.ds(k * CT, CT), :],
                o_hbm.at[b, pl.ds(k * CT, CT), :],
                out_sem.at[b],
            )
            c.start()
            outs.append(c)
    for c in outs:
        c.wait()


def _tc_part(W, B, T, H):
    K = 4
    CT = T // K
    body = functools.partial(_copy_body, B=B, K=K, CT=CT)
    return pl.pallas_call(
        body,
        in_specs=[pl.BlockSpec(memory_space=pl.ANY)],
        out_specs=pl.BlockSpec(memory_space=pl.ANY),
        out_shape=jax.ShapeDtypeStruct((B, T, H), W.dtype),
        scratch_shapes=[
            pltpu.VMEM((T, H), W.dtype),
            pltpu.SemaphoreType.DMA((K,)),
            pltpu.SemaphoreType.DMA((B,)),
        ],
    )(W)


def _sc_part(W, B, T, H):
    info = plsc.get_sparse_core_info()
    NC, NS = info.num_cores, info.num_subcores
    NW = NC * NS
    CT = T // NW

    mesh = plsc.VectorSubcoreMesh(core_axis_name="c", subcore_axis_name="s")

    @functools.partial(
        pl.kernel,
        mesh=mesh,
        out_type=jax.ShapeDtypeStruct((B, T, H), W.dtype),
        scratch_types=[
            pltpu.VMEM((CT, H), W.dtype),
            pltpu.SemaphoreType.DMA,
        ],
    )
    def bcast(w_hbm, out_hbm, w_v, sem):
        wid = lax.axis_index("s") * NC + lax.axis_index("c")
        base = wid * CT
        pltpu.sync_copy(w_hbm.at[pl.ds(base, CT), :], w_v)
        copies = [
            pltpu.make_async_copy(w_v, out_hbm.at[b, pl.ds(base, CT), :], sem)
            for b in range(B)
        ]
        for c in copies:
            c.start()
        for c in copies:
            c.wait()

    return bcast(W)


def kernel(x, W):
    B = x.shape[0]
    T, H = W.shape
    B_SC = 1
    tc = _tc_part(W, B - B_SC, T, H)
    sc = _sc_part(W, B_SC, T, H)
    return jnp.concatenate([tc, sc], axis=0)


# speedup vs baseline: 3.9763x; 3.9763x over previous
import functools

import jax
import jax.numpy as jnp
from jax.experimental import pallas as pl
from jax.experimental.pallas import tpu as pltpu


def _copy_body(w_hbm, o_hbm, w_vmem, in_sem, out_sem, *, B, KI, KO, CTI, CTO):
    ins = [
        pltpu.make_async_copy(
            w_hbm.at[pl.ds(k * CTI, CTI), :],
            w_vmem.at[pl.ds(k * CTI, CTI), :],
            in_sem.at[k],
        )
        for k in range(KI)
    ]
    for c in ins:
        c.start()
    r = KI // KO
    outs = []
    for k in range(KO):
        for j in range(r):
            ins[k * r + j].wait()
        for b in range(B):
            c = pltpu.make_async_copy(
                w_vmem.at[pl.ds(k * CTO, CTO), :],
                o_hbm.at[b, pl.ds(k * CTO, CTO), :],
                out_sem.at[b],
            )
            c.start()
            outs.append(c)
    for c in outs:
        c.wait()


def kernel(x, W):
    B = x.shape[0]
    T, H = W.shape
    KI = 4
    KO = 4
    CTI = T // KI
    CTO = T // KO
    body = functools.partial(_copy_body, B=B, KI=KI, KO=KO, CTI=CTI, CTO=CTO)
    return pl.pallas_call(
        body,
        in_specs=[pl.BlockSpec(memory_space=pl.ANY)],
        out_specs=pl.BlockSpec(memory_space=pl.ANY),
        out_shape=jax.ShapeDtypeStruct((B, T, H), W.dtype),
        scratch_shapes=[
            pltpu.VMEM((T, H), W.dtype),
            pltpu.SemaphoreType.DMA((KI,)),
            pltpu.SemaphoreType.DMA((B,)),
        ],
    )(W)
